# baseline (speedup 1.0000x reference)
import functools

import jax
import jax.numpy as jnp
from jax.experimental import pallas as pl
from jax.experimental.pallas import tpu as pltpu


def _router_kernel(x_ref, w1_ref, b1_ref, w2_ref, b2_ref, out_ref,
                   h_acc, logits_acc, *, n_steps, k_steps):
    n = pl.program_id(1)
    k = pl.program_id(2)

    part_h = jnp.dot(x_ref[...], w1_ref[...],
                     preferred_element_type=jnp.float32)

    @pl.when(k == 0)
    def _init_h():
        h_acc[...] = part_h

    @pl.when(k > 0)
    def _acc_h():
        h_acc[...] += part_h

    @pl.when(k == k_steps - 1)
    def _epilogue():
        h = h_acc[...] + b1_ref[...]
        hg = 0.5 * h * (1.0 + jax.lax.erf(h * 0.7071067811865476))
        part = jnp.dot(hg, w2_ref[...], preferred_element_type=jnp.float32)

        @pl.when(n == 0)
        def _init_logits():
            logits_acc[...] = part + b2_ref[...]

        @pl.when(n > 0)
        def _acc_logits():
            logits_acc[...] += part

        @pl.when(n == n_steps - 1)
        def _softmax():
            logits = logits_acc[...]
            mx = jnp.max(logits, axis=1, keepdims=True)
            e = jnp.exp(logits - mx)
            out_ref[...] = e / jnp.sum(e, axis=1, keepdims=True)


@jax.jit
def kernel(x, W1, b1, W2, b2):
    M, K = x.shape
    _, N = W1.shape
    E = W2.shape[1]

    bm = min(1024, M)
    bn = min(2048, N)
    bk = min(512, K)
    grid = (M // bm, N // bn, K // bk)

    b1r = b1.reshape(1, N)
    b2r = b2.reshape(1, E)

    return pl.pallas_call(
        functools.partial(_router_kernel, n_steps=grid[1], k_steps=grid[2]),
        grid=grid,
        in_specs=[
            pl.BlockSpec((bm, bk), lambda m, n, k: (m, k)),
            pl.BlockSpec((bk, bn), lambda m, n, k: (k, n)),
            pl.BlockSpec((1, bn), lambda m, n, k: (0, n)),
            pl.BlockSpec((bn, E), lambda m, n, k: (n, 0)),
            pl.BlockSpec((1, E), lambda m, n, k: (0, 0)),
        ],
        out_specs=pl.BlockSpec((bm, E), lambda m, n, k: (m, 0)),
        out_shape=jax.ShapeDtypeStruct((M, E), jnp.float32),
        scratch_shapes=[
            pltpu.VMEM((bm, bn), jnp.float32),
            pltpu.VMEM((bm, E), jnp.float32),
        ],
        compiler_params=pltpu.CompilerParams(
            dimension_semantics=("parallel", "arbitrary", "arbitrary")),
    )(x, W1, b1r, W2, b2r)


# speedup vs baseline: 1.3219x; 1.3219x over previous
import functools

import jax
import jax.numpy as jnp
from jax.experimental import pallas as pl
from jax.experimental.pallas import tpu as pltpu


def _router_kernel(x_ref, w1_ref, b1_ref, w2_ref, b2_ref, out_ref,
                   h_acc, logits_acc, *, n_steps, k_steps):
    n = pl.program_id(1)
    k = pl.program_id(2)

    @pl.when(k == 0)
    def _init_h():
        h_acc[...] = jnp.zeros_like(h_acc)

    h_acc[...] += jnp.dot(x_ref[...], w1_ref[...],
                          preferred_element_type=jnp.float32)

    @pl.when(k == k_steps - 1)
    def _epilogue():
        h = h_acc[...] + b1_ref[...]
        hg = 0.5 * h * (1.0 + jax.lax.erf(h * 0.7071067811865476))
        part = jnp.dot(hg, w2_ref[...], preferred_element_type=jnp.float32)

        @pl.when(n == 0)
        def _init_logits():
            logits_acc[...] = part + b2_ref[...]

        @pl.when(n > 0)
        def _acc_logits():
            logits_acc[...] += part

        @pl.when(n == n_steps - 1)
        def _softmax():
            logits = logits_acc[...]
            mx = jnp.max(logits, axis=1, keepdims=True)
            e = jnp.exp(logits - mx)
            out_ref[...] = e / jnp.sum(e, axis=1, keepdims=True)


@jax.jit
def kernel(x, W1, b1, W2, b2):
    M, K = x.shape
    _, N = W1.shape
    E = W2.shape[1]

    bm = min(1024, M)
    bn = min(4096, N)
    bk = min(512, K)
    grid = (M // bm, N // bn, K // bk)

    b1r = b1.reshape(1, N)
    b2r = b2.reshape(1, E)

    return pl.pallas_call(
        functools.partial(_router_kernel, n_steps=grid[1], k_steps=grid[2]),
        grid=grid,
        in_specs=[
            pl.BlockSpec((bm, bk), lambda m, n, k: (m, k)),
            pl.BlockSpec((bk, bn), lambda m, n, k: (k, n)),
            pl.BlockSpec((1, bn), lambda m, n, k: (0, n)),
            pl.BlockSpec((bn, E), lambda m, n, k: (n, 0)),
            pl.BlockSpec((1, E), lambda m, n, k: (0, 0)),
        ],
        out_specs=pl.BlockSpec((bm, E), lambda m, n, k: (m, 0)),
        out_shape=jax.ShapeDtypeStruct((M, E), jnp.float32),
        scratch_shapes=[
            pltpu.VMEM((bm, bn), jnp.float32),
            pltpu.VMEM((bm, E), jnp.float32),
        ],
        compiler_params=pltpu.CompilerParams(
            dimension_semantics=("parallel", "arbitrary", "arbitrary")),
    )(x, W1, b1r, W2, b2r)
